# baseline (speedup 1.0000x reference)
import functools

import jax
import jax.numpy as jnp
from jax.experimental import pallas as pl
from jax.experimental.pallas import tpu as pltpu

_F32 = jnp.float32
_BF16 = jnp.bfloat16


def _fold_bn(conv_w, g, b, m, v):
    scale = g * jax.lax.rsqrt(v + 1e-5)
    w = (conv_w * scale).astype(_F32)
    bias = (b - m * scale).astype(_F32)
    kh, kw, cin, cout = conv_w.shape
    return w.reshape(kh * kw * cin, cout), bias.reshape(1, cout)


def _store_xcol(xc_ref, v, nb, h, w, c):
    v = v.reshape(nb, h, w, c).astype(xc_ref.dtype)
    z = jnp.zeros((nb, 1, w, 3 * c), xc_ref.dtype)
    xc_ref[:, 0:1, :, :] = z
    xc_ref[:, h + 1:h + 2, :, :] = z
    xc_ref[:, 1:h + 1, 0:1, 0:c] = jnp.zeros((nb, h, 1, c), xc_ref.dtype)
    xc_ref[:, 1:h + 1, w - 1:w, 2 * c:3 * c] = jnp.zeros((nb, h, 1, c), xc_ref.dtype)
    xc_ref[:, 1:h + 1, 1:w, 0:c] = v[:, :, 0:w - 1, :]
    xc_ref[:, 1:h + 1, 0:w, c:2 * c] = v
    xc_ref[:, 1:h + 1, 0:w - 1, 2 * c:3 * c] = v[:, :, 1:w, :]


def _conv_xcol(xc_ref, w_ref, b_ref, nb, h, w, c, relu, extra=None):
    acc = None
    for i in range(3):
        a = xc_ref[:, i:i + h, :, :].reshape(nb * h * w, 3 * c)
        wt = w_ref[3 * i * c:3 * (i + 1) * c, :]
        d = jnp.dot(a, wt, preferred_element_type=_F32)
        acc = d if acc is None else acc + d
    acc = acc + b_ref[...]
    if extra is not None:
        acc = acc + extra
    if relu:
        acc = jnp.maximum(acc, 0.0)
    return acc


def _store_xcp(xcp_ref, x, nb, h, w, c):
    ho, wo = h // 2, w // 2
    x4 = x.reshape(nb, ho, 2, wo, 2, c).astype(xcp_ref.dtype)
    xcp_ref[1, :, ho:ho + 1, :, :] = jnp.zeros((nb, 1, wo, 3 * c), xcp_ref.dtype)
    xcp_ref[0, :, 0:1, :, :] = jnp.zeros((nb, 1, wo, 3 * c), xcp_ref.dtype)
    xcp_ref[1, :, 0:ho, 0:1, 0:c] = jnp.zeros((nb, ho, 1, c), xcp_ref.dtype)
    xcp_ref[0, :, 1:ho + 1, 0:1, 0:c] = jnp.zeros((nb, ho, 1, c), xcp_ref.dtype)
    xcp_ref[1, :, 0:ho, 1:wo, 0:c] = x4[:, :, 0, 0:wo - 1, 1, :]
    xcp_ref[1, :, 0:ho, :, c:2 * c] = x4[:, :, 0, :, 0, :]
    xcp_ref[1, :, 0:ho, :, 2 * c:3 * c] = x4[:, :, 0, :, 1, :]
    xcp_ref[0, :, 1:ho + 1, 1:wo, 0:c] = x4[:, :, 1, 0:wo - 1, 1, :]
    xcp_ref[0, :, 1:ho + 1, :, c:2 * c] = x4[:, :, 1, :, 0, :]
    xcp_ref[0, :, 1:ho + 1, :, 2 * c:3 * c] = x4[:, :, 1, :, 1, :]


def _conv_xcp(xcp_ref, w_ref, b_ref, nb, ho, wo, c):
    acc = None
    for i in range(3):
        a = xcp_ref[i % 2, :, i // 2:i // 2 + ho, :, :].reshape(
            nb * ho * wo, 3 * c)
        wt = w_ref[3 * i * c:3 * (i + 1) * c, :]
        d = jnp.dot(a, wt, preferred_element_type=_F32)
        acc = d if acc is None else acc + d
    acc = jnp.maximum(acc + b_ref[...], 0.0)
    x_even = xcp_ref[1, :, 0:ho, :, c:2 * c].reshape(nb * ho * wo, c)
    return acc, x_even


def _stage1_body(xp_ref, w0, b0, w1a, b1a, w1b, b1b, w2a, b2a, w2b, b2b,
                 o_ref, xcA, xcB, *, nb):
    h = w = 32
    c = 64
    x0 = xp_ref[...].reshape(nb * h * w, 27)
    v0 = jnp.maximum(jnp.dot(x0, w0[...], preferred_element_type=_F32)
                     + b0[...], 0.0)
    _store_xcol(xcA, v0, nb, h, w, c)
    v1 = _conv_xcol(xcA, w1a, b1a, nb, h, w, c, relu=True)
    _store_xcol(xcB, v1, nb, h, w, c)
    v2 = _conv_xcol(xcB, w1b, b1b, nb, h, w, c, relu=True, extra=v0)
    _store_xcol(xcA, v2, nb, h, w, c)
    v3 = _conv_xcol(xcA, w2a, b2a, nb, h, w, c, relu=True)
    _store_xcol(xcB, v3, nb, h, w, c)
    v4 = _conv_xcol(xcB, w2b, b2b, nb, h, w, c, relu=True, extra=v2)
    o_ref[...] = v4.reshape(nb, h, w, c).astype(o_ref.dtype)


def _down_body(x_ref, wd, bd, w1a, b1a, w1b, b1b, w2a, b2a, w2b, b2b,
               o_ref, xcp, xcB, *, nb, h, cin, cout):
    ho = h // 2
    _store_xcp(xcp, x_ref[...], nb, h, h, cin)
    v1, x_even = _conv_xcp(xcp, w1a, b1a, nb, ho, ho, cin)
    sc = jnp.dot(x_even, wd[...], preferred_element_type=_F32) + bd[...]
    _store_xcol(xcB, v1, nb, ho, ho, cout)
    v2 = _conv_xcol(xcB, w1b, b1b, nb, ho, ho, cout, relu=True, extra=sc)
    _store_xcol(xcB, v2, nb, ho, ho, cout)
    v3 = _conv_xcol(xcB, w2a, b2a, nb, ho, ho, cout, relu=True)
    _store_xcol(xcB, v3, nb, ho, ho, cout)
    v4 = _conv_xcol(xcB, w2b, b2b, nb, ho, ho, cout, relu=True, extra=v2)
    o_ref[...] = v4.reshape(nb, ho, ho, cout).astype(o_ref.dtype)


def _l4b0_body(x_ref, wd, bd, w1a, b1a, w1b, b1b, o_ref, xcp, xcB, *, nb):
    h, cin, cout = 8, 256, 512
    ho = h // 2
    _store_xcp(xcp, x_ref[...], nb, h, h, cin)
    v1, x_even = _conv_xcp(xcp, w1a, b1a, nb, ho, ho, cin)
    sc = jnp.dot(x_even, wd[...], preferred_element_type=_F32) + bd[...]
    _store_xcol(xcB, v1, nb, ho, ho, cout)
    v2 = _conv_xcol(xcB, w1b, b1b, nb, ho, ho, cout, relu=True, extra=sc)
    o_ref[...] = v2.reshape(nb, ho, ho, cout).astype(o_ref.dtype)


def _mega_body(xp_ref,
               w0, b0, w11a, b11a, w11b, b11b, w12a, b12a, w12b, b12b,
               w2d, b2d, w21a, b21a, w21b, b21b, w22a, b22a, w22b, b22b,
               w3d, b3d, w31a, b31a, w31b, b31b, w32a, b32a, w32b, b32b,
               w4d, b4d, w41a, b41a, w41b, b41b,
               w42a, b42a, w42b, b42b, wfc, bfc,
               o_ref, xc1, xcp2, xc2, xcp3, xc3, xcp4, xc4, *, nb):
    x0 = xp_ref[...].reshape(nb * 1024, 27)
    v0 = jnp.maximum(jnp.dot(x0, w0[...], preferred_element_type=_F32)
                     + b0[...], 0.0)
    _store_xcol(xc1, v0, nb, 32, 32, 64)
    v1 = _conv_xcol(xc1, w11a, b11a, nb, 32, 32, 64, relu=True)
    _store_xcol(xc1, v1, nb, 32, 32, 64)
    v2 = _conv_xcol(xc1, w11b, b11b, nb, 32, 32, 64, relu=True, extra=v0)
    _store_xcol(xc1, v2, nb, 32, 32, 64)
    v3 = _conv_xcol(xc1, w12a, b12a, nb, 32, 32, 64, relu=True)
    _store_xcol(xc1, v3, nb, 32, 32, 64)
    h = _conv_xcol(xc1, w12b, b12b, nb, 32, 32, 64, relu=True, extra=v2)
    _store_xcp(xcp2, h, nb, 32, 32, 64)
    v1, xe = _conv_xcp(xcp2, w21a, b21a, nb, 16, 16, 64)
    sc = jnp.dot(xe, w2d[...], preferred_element_type=_F32) + b2d[...]
    _store_xcol(xc2, v1, nb, 16, 16, 128)
    v2 = _conv_xcol(xc2, w21b, b21b, nb, 16, 16, 128, relu=True, extra=sc)
    _store_xcol(xc2, v2, nb, 16, 16, 128)
    v3 = _conv_xcol(xc2, w22a, b22a, nb, 16, 16, 128, relu=True)
    _store_xcol(xc2, v3, nb, 16, 16, 128)
    h = _conv_xcol(xc2, w22b, b22b, nb, 16, 16, 128, relu=True, extra=v2)
    _store_xcp(xcp3, h, nb, 16, 16, 128)
    v1, xe = _conv_xcp(xcp3, w31a, b31a, nb, 8, 8, 128)
    sc = jnp.dot(xe, w3d[...], preferred_element_type=_F32) + b3d[...]
    _store_xcol(xc3, v1, nb, 8, 8, 256)
    v2 = _conv_xcol(xc3, w31b, b31b, nb, 8, 8, 256, relu=True, extra=sc)
    _store_xcol(xc3, v2, nb, 8, 8, 256)
    v3 = _conv_xcol(xc3, w32a, b32a, nb, 8, 8, 256, relu=True)
    _store_xcol(xc3, v3, nb, 8, 8, 256)
    h = _conv_xcol(xc3, w32b, b32b, nb, 8, 8, 256, relu=True, extra=v2)
    _store_xcp(xcp4, h, nb, 8, 8, 256)
    v1, xe = _conv_xcp(xcp4, w41a, b41a, nb, 4, 4, 256)
    sc = jnp.dot(xe, w4d[...], preferred_element_type=_F32) + b4d[...]
    _store_xcol(xc4, v1, nb, 4, 4, 512)
    x0 = _conv_xcol(xc4, w41b, b41b, nb, 4, 4, 512, relu=True, extra=sc)
    _store_xcol(xc4, x0, nb, 4, 4, 512)
    v1 = _conv_xcol(xc4, w42a, b42a, nb, 4, 4, 512, relu=True)
    _store_xcol(xc4, v1, nb, 4, 4, 512)
    v2 = _conv_xcol(xc4, w42b, b42b, nb, 4, 4, 512, relu=True, extra=x0)
    pooled = v2.reshape(nb, 16, 512).sum(axis=1) * _F32(1.0 / 16.0)
    logits = jnp.dot(pooled, wfc[...], preferred_element_type=_F32) + bfc[...]
    m = jnp.max(logits, axis=1, keepdims=True)
    sh = logits - m
    lse = jnp.log(jnp.sum(jnp.exp(sh), axis=1, keepdims=True))
    o_ref[...] = (sh - lse).reshape(o_ref.shape)


def _l4b1_head_body(x_ref, w1a, b1a, w1b, b1b, wfc, bfc, o_ref, xcB, *, nb):
    h, c = 4, 512
    x0 = x_ref[...].reshape(nb * h * h, c)
    _store_xcol(xcB, x0, nb, h, h, c)
    v1 = _conv_xcol(xcB, w1a, b1a, nb, h, h, c, relu=True)
    _store_xcol(xcB, v1, nb, h, h, c)
    v2 = _conv_xcol(xcB, w1b, b1b, nb, h, h, c, relu=True, extra=x0)
    pooled = v2.reshape(nb, h * h, c).sum(axis=1) * _F32(1.0 / (h * h))
    logits = jnp.dot(pooled, wfc[...], preferred_element_type=_F32) + bfc[...]
    m = jnp.max(logits, axis=1, keepdims=True)
    sh = logits - m
    lse = jnp.log(jnp.sum(jnp.exp(sh), axis=1, keepdims=True))
    o_ref[...] = sh - lse


_PARAMS = pltpu.CompilerParams(dimension_semantics=("parallel",),
                               vmem_limit_bytes=64 * 1024 * 1024)


def _full(arr_ndim):
    return lambda i: (0,) * arr_ndim


def _run_stage(body, grid, in_arrays, in_blocks, out_shape, out_block,
               scratches):
    in_specs = []
    for arr, blk in zip(in_arrays, in_blocks):
        if blk is None:
            in_specs.append(pl.BlockSpec(arr.shape, _full(arr.ndim)))
        else:
            idx = (lambda nd: lambda i: (i,) + (0,) * (nd - 1))(len(blk))
            in_specs.append(pl.BlockSpec(blk, idx))
    return pl.pallas_call(
        body,
        grid=(grid,),
        in_specs=in_specs,
        out_specs=pl.BlockSpec(out_block,
                               lambda i: (i,) + (0,) * (len(out_block) - 1)),
        out_shape=out_shape,
        scratch_shapes=scratches,
        compiler_params=_PARAMS,
    )(*in_arrays)


def kernel(x, conv1_w, bn1_g, bn1_b, bn1_m, bn1_v, l1b0_conv1_w, l1b0_bn1_g, l1b0_bn1_b, l1b0_bn1_m, l1b0_bn1_v, l1b0_conv2_w, l1b0_bn2_g, l1b0_bn2_b, l1b0_bn2_m, l1b0_bn2_v, l1b1_conv1_w, l1b1_bn1_g, l1b1_bn1_b, l1b1_bn1_m, l1b1_bn1_v, l1b1_conv2_w, l1b1_bn2_g, l1b1_bn2_b, l1b1_bn2_m, l1b1_bn2_v, l2b0_conv1_w, l2b0_bn1_g, l2b0_bn1_b, l2b0_bn1_m, l2b0_bn1_v, l2b0_conv2_w, l2b0_bn2_g, l2b0_bn2_b, l2b0_bn2_m, l2b0_bn2_v, l2b0_down_w, l2b0_down_bn_g, l2b0_down_bn_b, l2b0_down_bn_m, l2b0_down_bn_v, l2b1_conv1_w, l2b1_bn1_g, l2b1_bn1_b, l2b1_bn1_m, l2b1_bn1_v, l2b1_conv2_w, l2b1_bn2_g, l2b1_bn2_b, l2b1_bn2_m, l2b1_bn2_v, l3b0_conv1_w, l3b0_bn1_g, l3b0_bn1_b, l3b0_bn1_m, l3b0_bn1_v, l3b0_conv2_w, l3b0_bn2_g, l3b0_bn2_b, l3b0_bn2_m, l3b0_bn2_v, l3b0_down_w, l3b0_down_bn_g, l3b0_down_bn_b, l3b0_down_bn_m, l3b0_down_bn_v, l3b1_conv1_w, l3b1_bn1_g, l3b1_bn1_b, l3b1_bn1_m, l3b1_bn1_v, l3b1_conv2_w, l3b1_bn2_g, l3b1_bn2_b, l3b1_bn2_m, l3b1_bn2_v, l4b0_conv1_w, l4b0_bn1_g, l4b0_bn1_b, l4b0_bn1_m, l4b0_bn1_v, l4b0_conv2_w, l4b0_bn2_g, l4b0_bn2_b, l4b0_bn2_m, l4b0_bn2_v, l4b0_down_w, l4b0_down_bn_g, l4b0_down_bn_b, l4b0_down_bn_m, l4b0_down_bn_v, l4b1_conv1_w, l4b1_bn1_g, l4b1_bn1_b, l4b1_bn1_m, l4b1_bn1_v, l4b1_conv2_w, l4b1_bn2_g, l4b1_bn2_b, l4b1_bn2_m, l4b1_bn2_v, fc_w, fc_b):
    n = x.shape[0]

    w0, b0 = _fold_bn(conv1_w, bn1_g, bn1_b, bn1_m, bn1_v)
    w11a, b11a = _fold_bn(l1b0_conv1_w, l1b0_bn1_g, l1b0_bn1_b, l1b0_bn1_m, l1b0_bn1_v)
    w11b, b11b = _fold_bn(l1b0_conv2_w, l1b0_bn2_g, l1b0_bn2_b, l1b0_bn2_m, l1b0_bn2_v)
    w12a, b12a = _fold_bn(l1b1_conv1_w, l1b1_bn1_g, l1b1_bn1_b, l1b1_bn1_m, l1b1_bn1_v)
    w12b, b12b = _fold_bn(l1b1_conv2_w, l1b1_bn2_g, l1b1_bn2_b, l1b1_bn2_m, l1b1_bn2_v)
    w21a, b21a = _fold_bn(l2b0_conv1_w, l2b0_bn1_g, l2b0_bn1_b, l2b0_bn1_m, l2b0_bn1_v)
    w21b, b21b = _fold_bn(l2b0_conv2_w, l2b0_bn2_g, l2b0_bn2_b, l2b0_bn2_m, l2b0_bn2_v)
    w2d, b2d = _fold_bn(l2b0_down_w, l2b0_down_bn_g, l2b0_down_bn_b, l2b0_down_bn_m, l2b0_down_bn_v)
    w22a, b22a = _fold_bn(l2b1_conv1_w, l2b1_bn1_g, l2b1_bn1_b, l2b1_bn1_m, l2b1_bn1_v)
    w22b, b22b = _fold_bn(l2b1_conv2_w, l2b1_bn2_g, l2b1_bn2_b, l2b1_bn2_m, l2b1_bn2_v)
    w31a, b31a = _fold_bn(l3b0_conv1_w, l3b0_bn1_g, l3b0_bn1_b, l3b0_bn1_m, l3b0_bn1_v)
    w31b, b31b = _fold_bn(l3b0_conv2_w, l3b0_bn2_g, l3b0_bn2_b, l3b0_bn2_m, l3b0_bn2_v)
    w3d, b3d = _fold_bn(l3b0_down_w, l3b0_down_bn_g, l3b0_down_bn_b, l3b0_down_bn_m, l3b0_down_bn_v)
    w32a, b32a = _fold_bn(l3b1_conv1_w, l3b1_bn1_g, l3b1_bn1_b, l3b1_bn1_m, l3b1_bn1_v)
    w32b, b32b = _fold_bn(l3b1_conv2_w, l3b1_bn2_g, l3b1_bn2_b, l3b1_bn2_m, l3b1_bn2_v)
    w41a, b41a = _fold_bn(l4b0_conv1_w, l4b0_bn1_g, l4b0_bn1_b, l4b0_bn1_m, l4b0_bn1_v)
    w41b, b41b = _fold_bn(l4b0_conv2_w, l4b0_bn2_g, l4b0_bn2_b, l4b0_bn2_m, l4b0_bn2_v)
    w4d, b4d = _fold_bn(l4b0_down_w, l4b0_down_bn_g, l4b0_down_bn_b, l4b0_down_bn_m, l4b0_down_bn_v)
    w42a, b42a = _fold_bn(l4b1_conv1_w, l4b1_bn1_g, l4b1_bn1_b, l4b1_bn1_m, l4b1_bn1_v)
    w42b, b42b = _fold_bn(l4b1_conv2_w, l4b1_bn2_g, l4b1_bn2_b, l4b1_bn2_m, l4b1_bn2_v)

    ncls = fc_w.shape[1]
    ncls_p = 128
    wfc = jnp.pad(fc_w.astype(_F32), ((0, 0), (0, ncls_p - ncls)))
    bfc = jnp.concatenate(
        [fc_b.astype(_F32),
         jnp.full((ncls_p - ncls,), -1e30, _F32)]).reshape(1, ncls_p)

    xp = jnp.pad(x, ((0, 0), (1, 1), (1, 1), (0, 0)))
    cols = [xp[:, i:i + 32, j:j + 32, :] for i in range(3) for j in range(3)]
    x27 = jnp.concatenate(cols, axis=-1)

    nb = 4
    out = _run_stage(
        functools.partial(_mega_body, nb=nb), n // nb,
        [x27,
         w0, b0, w11a, b11a, w11b, b11b, w12a, b12a, w12b, b12b,
         w2d, b2d, w21a, b21a, w21b, b21b, w22a, b22a, w22b, b22b,
         w3d, b3d, w31a, b31a, w31b, b31b, w32a, b32a, w32b, b32b,
         w4d, b4d, w41a, b41a, w41b, b41b,
         w42a, b42a, w42b, b42b, wfc, bfc],
        [(nb, 32, 32, 27)] + [None] * 42,
        jax.ShapeDtypeStruct((n // nb, nb, ncls_p), _F32), (1, nb, ncls_p),
        [pltpu.VMEM((nb, 34, 32, 192), _F32),
         pltpu.VMEM((2, nb, 17, 16, 192), _F32),
         pltpu.VMEM((nb, 18, 16, 384), _F32),
         pltpu.VMEM((2, nb, 9, 8, 384), _F32),
         pltpu.VMEM((nb, 10, 8, 768), _F32),
         pltpu.VMEM((2, nb, 5, 4, 768), _F32),
         pltpu.VMEM((nb, 6, 4, 1536), _F32)])

    return out.reshape(n, ncls_p)[:, :ncls]


# speedup vs baseline: 1.3041x; 1.3041x over previous
import functools

import jax
import jax.numpy as jnp
import numpy as np
from jax.experimental import pallas as pl
from jax.experimental.pallas import tpu as pltpu
from jax.experimental.shard_map import shard_map
from jax.sharding import Mesh, NamedSharding, PartitionSpec as P

_F32 = jnp.float32
_BF16 = jnp.bfloat16


def _fold_bn(conv_w, g, b, m, v):
    scale = g * jax.lax.rsqrt(v + 1e-5)
    w = (conv_w * scale).astype(_F32)
    bias = (b - m * scale).astype(_F32)
    kh, kw, cin, cout = conv_w.shape
    return w.reshape(kh * kw * cin, cout), bias.reshape(1, cout)


def _store_xcol(xc_ref, v, nb, h, w, c):
    v = v.reshape(nb, h, w, c).astype(xc_ref.dtype)
    z = jnp.zeros((nb, 1, w, 3 * c), xc_ref.dtype)
    xc_ref[:, 0:1, :, :] = z
    xc_ref[:, h + 1:h + 2, :, :] = z
    xc_ref[:, 1:h + 1, 0:1, 0:c] = jnp.zeros((nb, h, 1, c), xc_ref.dtype)
    xc_ref[:, 1:h + 1, w - 1:w, 2 * c:3 * c] = jnp.zeros((nb, h, 1, c), xc_ref.dtype)
    xc_ref[:, 1:h + 1, 1:w, 0:c] = v[:, :, 0:w - 1, :]
    xc_ref[:, 1:h + 1, 0:w, c:2 * c] = v
    xc_ref[:, 1:h + 1, 0:w - 1, 2 * c:3 * c] = v[:, :, 1:w, :]


def _conv_xcol(xc_ref, w_ref, b_ref, nb, h, w, c, relu, extra=None):
    acc = None
    for i in range(3):
        a = xc_ref[:, i:i + h, :, :].reshape(nb * h * w, 3 * c)
        wt = w_ref[3 * i * c:3 * (i + 1) * c, :]
        d = jnp.dot(a, wt, preferred_element_type=_F32)
        acc = d if acc is None else acc + d
    acc = acc + b_ref[...]
    if extra is not None:
        acc = acc + extra
    if relu:
        acc = jnp.maximum(acc, 0.0)
    return acc


def _store_xcp(xcp_ref, x, nb, h, w, c):
    ho, wo = h // 2, w // 2
    x4 = x.reshape(nb, ho, 2, wo, 2, c).astype(xcp_ref.dtype)
    xcp_ref[1, :, ho:ho + 1, :, :] = jnp.zeros((nb, 1, wo, 3 * c), xcp_ref.dtype)
    xcp_ref[0, :, 0:1, :, :] = jnp.zeros((nb, 1, wo, 3 * c), xcp_ref.dtype)
    xcp_ref[1, :, 0:ho, 0:1, 0:c] = jnp.zeros((nb, ho, 1, c), xcp_ref.dtype)
    xcp_ref[0, :, 1:ho + 1, 0:1, 0:c] = jnp.zeros((nb, ho, 1, c), xcp_ref.dtype)
    xcp_ref[1, :, 0:ho, 1:wo, 0:c] = x4[:, :, 0, 0:wo - 1, 1, :]
    xcp_ref[1, :, 0:ho, :, c:2 * c] = x4[:, :, 0, :, 0, :]
    xcp_ref[1, :, 0:ho, :, 2 * c:3 * c] = x4[:, :, 0, :, 1, :]
    xcp_ref[0, :, 1:ho + 1, 1:wo, 0:c] = x4[:, :, 1, 0:wo - 1, 1, :]
    xcp_ref[0, :, 1:ho + 1, :, c:2 * c] = x4[:, :, 1, :, 0, :]
    xcp_ref[0, :, 1:ho + 1, :, 2 * c:3 * c] = x4[:, :, 1, :, 1, :]


def _conv_xcp(xcp_ref, w_ref, b_ref, nb, ho, wo, c):
    acc = None
    for i in range(3):
        a = xcp_ref[i % 2, :, i // 2:i // 2 + ho, :, :].reshape(
            nb * ho * wo, 3 * c)
        wt = w_ref[3 * i * c:3 * (i + 1) * c, :]
        d = jnp.dot(a, wt, preferred_element_type=_F32)
        acc = d if acc is None else acc + d
    acc = jnp.maximum(acc + b_ref[...], 0.0)
    x_even = xcp_ref[1, :, 0:ho, :, c:2 * c].reshape(nb * ho * wo, c)
    return acc, x_even


def _stage1_body(xp_ref, w0, b0, w1a, b1a, w1b, b1b, w2a, b2a, w2b, b2b,
                 o_ref, xcA, xcB, *, nb):
    h = w = 32
    c = 64
    x0 = xp_ref[...].reshape(nb * h * w, 27)
    v0 = jnp.maximum(jnp.dot(x0, w0[...], preferred_element_type=_F32)
                     + b0[...], 0.0)
    _store_xcol(xcA, v0, nb, h, w, c)
    v1 = _conv_xcol(xcA, w1a, b1a, nb, h, w, c, relu=True)
    _store_xcol(xcB, v1, nb, h, w, c)
    v2 = _conv_xcol(xcB, w1b, b1b, nb, h, w, c, relu=True, extra=v0)
    _store_xcol(xcA, v2, nb, h, w, c)
    v3 = _conv_xcol(xcA, w2a, b2a, nb, h, w, c, relu=True)
    _store_xcol(xcB, v3, nb, h, w, c)
    v4 = _conv_xcol(xcB, w2b, b2b, nb, h, w, c, relu=True, extra=v2)
    o_ref[...] = v4.reshape(nb, h, w, c).astype(o_ref.dtype)


def _down_body(x_ref, wd, bd, w1a, b1a, w1b, b1b, w2a, b2a, w2b, b2b,
               o_ref, xcp, xcB, *, nb, h, cin, cout):
    ho = h // 2
    _store_xcp(xcp, x_ref[...], nb, h, h, cin)
    v1, x_even = _conv_xcp(xcp, w1a, b1a, nb, ho, ho, cin)
    sc = jnp.dot(x_even, wd[...], preferred_element_type=_F32) + bd[...]
    _store_xcol(xcB, v1, nb, ho, ho, cout)
    v2 = _conv_xcol(xcB, w1b, b1b, nb, ho, ho, cout, relu=True, extra=sc)
    _store_xcol(xcB, v2, nb, ho, ho, cout)
    v3 = _conv_xcol(xcB, w2a, b2a, nb, ho, ho, cout, relu=True)
    _store_xcol(xcB, v3, nb, ho, ho, cout)
    v4 = _conv_xcol(xcB, w2b, b2b, nb, ho, ho, cout, relu=True, extra=v2)
    o_ref[...] = v4.reshape(nb, ho, ho, cout).astype(o_ref.dtype)


def _l4b0_body(x_ref, wd, bd, w1a, b1a, w1b, b1b, o_ref, xcp, xcB, *, nb):
    h, cin, cout = 8, 256, 512
    ho = h // 2
    _store_xcp(xcp, x_ref[...], nb, h, h, cin)
    v1, x_even = _conv_xcp(xcp, w1a, b1a, nb, ho, ho, cin)
    sc = jnp.dot(x_even, wd[...], preferred_element_type=_F32) + bd[...]
    _store_xcol(xcB, v1, nb, ho, ho, cout)
    v2 = _conv_xcol(xcB, w1b, b1b, nb, ho, ho, cout, relu=True, extra=sc)
    o_ref[...] = v2.reshape(nb, ho, ho, cout).astype(o_ref.dtype)


def _l4b1_head_body(x_ref, w1a, b1a, w1b, b1b, wfc, bfc, o_ref, xcB, *, nb):
    h, c = 4, 512
    x0 = x_ref[...].reshape(nb * h * h, c)
    _store_xcol(xcB, x0, nb, h, h, c)
    v1 = _conv_xcol(xcB, w1a, b1a, nb, h, h, c, relu=True)
    _store_xcol(xcB, v1, nb, h, h, c)
    v2 = _conv_xcol(xcB, w1b, b1b, nb, h, h, c, relu=True, extra=x0)
    pooled = v2.reshape(nb, h * h, c).sum(axis=1) * _F32(1.0 / (h * h))
    logits = jnp.dot(pooled, wfc[...], preferred_element_type=_F32) + bfc[...]
    m = jnp.max(logits, axis=1, keepdims=True)
    sh = logits - m
    lse = jnp.log(jnp.sum(jnp.exp(sh), axis=1, keepdims=True))
    o_ref[...] = sh - lse


_PARAMS = pltpu.CompilerParams(dimension_semantics=("parallel",),
                               vmem_limit_bytes=60 * 1024 * 1024)


def _full(arr_ndim):
    return lambda i: (0,) * arr_ndim


def _run_stage(body, grid, in_arrays, in_blocks, out_shape, out_block,
               scratches):
    in_specs = []
    for arr, blk in zip(in_arrays, in_blocks):
        if blk is None:
            in_specs.append(pl.BlockSpec(arr.shape, _full(arr.ndim)))
        else:
            idx = (lambda nd: lambda i: (i,) + (0,) * (nd - 1))(len(blk))
            in_specs.append(pl.BlockSpec(blk, idx))
    return pl.pallas_call(
        body,
        grid=(grid,),
        in_specs=in_specs,
        out_specs=pl.BlockSpec(out_block,
                               lambda i: (i,) + (0,) * (len(out_block) - 1)),
        out_shape=out_shape,
        scratch_shapes=scratches,
        compiler_params=_PARAMS,
    )(*in_arrays)


def kernel(x, conv1_w, bn1_g, bn1_b, bn1_m, bn1_v, l1b0_conv1_w, l1b0_bn1_g, l1b0_bn1_b, l1b0_bn1_m, l1b0_bn1_v, l1b0_conv2_w, l1b0_bn2_g, l1b0_bn2_b, l1b0_bn2_m, l1b0_bn2_v, l1b1_conv1_w, l1b1_bn1_g, l1b1_bn1_b, l1b1_bn1_m, l1b1_bn1_v, l1b1_conv2_w, l1b1_bn2_g, l1b1_bn2_b, l1b1_bn2_m, l1b1_bn2_v, l2b0_conv1_w, l2b0_bn1_g, l2b0_bn1_b, l2b0_bn1_m, l2b0_bn1_v, l2b0_conv2_w, l2b0_bn2_g, l2b0_bn2_b, l2b0_bn2_m, l2b0_bn2_v, l2b0_down_w, l2b0_down_bn_g, l2b0_down_bn_b, l2b0_down_bn_m, l2b0_down_bn_v, l2b1_conv1_w, l2b1_bn1_g, l2b1_bn1_b, l2b1_bn1_m, l2b1_bn1_v, l2b1_conv2_w, l2b1_bn2_g, l2b1_bn2_b, l2b1_bn2_m, l2b1_bn2_v, l3b0_conv1_w, l3b0_bn1_g, l3b0_bn1_b, l3b0_bn1_m, l3b0_bn1_v, l3b0_conv2_w, l3b0_bn2_g, l3b0_bn2_b, l3b0_bn2_m, l3b0_bn2_v, l3b0_down_w, l3b0_down_bn_g, l3b0_down_bn_b, l3b0_down_bn_m, l3b0_down_bn_v, l3b1_conv1_w, l3b1_bn1_g, l3b1_bn1_b, l3b1_bn1_m, l3b1_bn1_v, l3b1_conv2_w, l3b1_bn2_g, l3b1_bn2_b, l3b1_bn2_m, l3b1_bn2_v, l4b0_conv1_w, l4b0_bn1_g, l4b0_bn1_b, l4b0_bn1_m, l4b0_bn1_v, l4b0_conv2_w, l4b0_bn2_g, l4b0_bn2_b, l4b0_bn2_m, l4b0_bn2_v, l4b0_down_w, l4b0_down_bn_g, l4b0_down_bn_b, l4b0_down_bn_m, l4b0_down_bn_v, l4b1_conv1_w, l4b1_bn1_g, l4b1_bn1_b, l4b1_bn1_m, l4b1_bn1_v, l4b1_conv2_w, l4b1_bn2_g, l4b1_bn2_b, l4b1_bn2_m, l4b1_bn2_v, fc_w, fc_b):
    n = x.shape[0]

    w0, b0 = _fold_bn(conv1_w, bn1_g, bn1_b, bn1_m, bn1_v)
    w11a, b11a = _fold_bn(l1b0_conv1_w, l1b0_bn1_g, l1b0_bn1_b, l1b0_bn1_m, l1b0_bn1_v)
    w11b, b11b = _fold_bn(l1b0_conv2_w, l1b0_bn2_g, l1b0_bn2_b, l1b0_bn2_m, l1b0_bn2_v)
    w12a, b12a = _fold_bn(l1b1_conv1_w, l1b1_bn1_g, l1b1_bn1_b, l1b1_bn1_m, l1b1_bn1_v)
    w12b, b12b = _fold_bn(l1b1_conv2_w, l1b1_bn2_g, l1b1_bn2_b, l1b1_bn2_m, l1b1_bn2_v)
    w21a, b21a = _fold_bn(l2b0_conv1_w, l2b0_bn1_g, l2b0_bn1_b, l2b0_bn1_m, l2b0_bn1_v)
    w21b, b21b = _fold_bn(l2b0_conv2_w, l2b0_bn2_g, l2b0_bn2_b, l2b0_bn2_m, l2b0_bn2_v)
    w2d, b2d = _fold_bn(l2b0_down_w, l2b0_down_bn_g, l2b0_down_bn_b, l2b0_down_bn_m, l2b0_down_bn_v)
    w22a, b22a = _fold_bn(l2b1_conv1_w, l2b1_bn1_g, l2b1_bn1_b, l2b1_bn1_m, l2b1_bn1_v)
    w22b, b22b = _fold_bn(l2b1_conv2_w, l2b1_bn2_g, l2b1_bn2_b, l2b1_bn2_m, l2b1_bn2_v)
    w31a, b31a = _fold_bn(l3b0_conv1_w, l3b0_bn1_g, l3b0_bn1_b, l3b0_bn1_m, l3b0_bn1_v)
    w31b, b31b = _fold_bn(l3b0_conv2_w, l3b0_bn2_g, l3b0_bn2_b, l3b0_bn2_m, l3b0_bn2_v)
    w3d, b3d = _fold_bn(l3b0_down_w, l3b0_down_bn_g, l3b0_down_bn_b, l3b0_down_bn_m, l3b0_down_bn_v)
    w32a, b32a = _fold_bn(l3b1_conv1_w, l3b1_bn1_g, l3b1_bn1_b, l3b1_bn1_m, l3b1_bn1_v)
    w32b, b32b = _fold_bn(l3b1_conv2_w, l3b1_bn2_g, l3b1_bn2_b, l3b1_bn2_m, l3b1_bn2_v)
    w41a, b41a = _fold_bn(l4b0_conv1_w, l4b0_bn1_g, l4b0_bn1_b, l4b0_bn1_m, l4b0_bn1_v)
    w41b, b41b = _fold_bn(l4b0_conv2_w, l4b0_bn2_g, l4b0_bn2_b, l4b0_bn2_m, l4b0_bn2_v)
    w4d, b4d = _fold_bn(l4b0_down_w, l4b0_down_bn_g, l4b0_down_bn_b, l4b0_down_bn_m, l4b0_down_bn_v)
    w42a, b42a = _fold_bn(l4b1_conv1_w, l4b1_bn1_g, l4b1_bn1_b, l4b1_bn1_m, l4b1_bn1_v)
    w42b, b42b = _fold_bn(l4b1_conv2_w, l4b1_bn2_g, l4b1_bn2_b, l4b1_bn2_m, l4b1_bn2_v)

    ncls = fc_w.shape[1]
    ncls_p = 128
    wfc = jnp.pad(fc_w.astype(_F32), ((0, 0), (0, ncls_p - ncls)))
    bfc = jnp.concatenate(
        [fc_b.astype(_F32),
         jnp.full((ncls_p - ncls,), -1e30, _F32)]).reshape(1, ncls_p)

    xp = jnp.pad(x, ((0, 0), (1, 1), (1, 1), (0, 0)))
    cols = [xp[:, i:i + 32, j:j + 32, :] for i in range(3) for j in range(3)]
    x27 = jnp.concatenate(cols, axis=-1)

    def _forward(x27, w0, b0, w11a, b11a, w11b, b11b, w12a, b12a, w12b, b12b,
                 w2d, b2d, w21a, b21a, w21b, b21b, w22a, b22a, w22b, b22b,
                 w3d, b3d, w31a, b31a, w31b, b31b, w32a, b32a, w32b, b32b,
                 w4d, b4d, w41a, b41a, w41b, b41b,
                 w42a, b42a, w42b, b42b, wfc, bfc):
        nloc = x27.shape[0]
        nb = 8
        h1 = _run_stage(
            functools.partial(_stage1_body, nb=nb), nloc // nb,
            [x27, w0, b0, w11a, b11a, w11b, b11b, w12a, b12a, w12b, b12b],
            [(nb, 32, 32, 27)] + [None] * 10,
            jax.ShapeDtypeStruct((nloc, 32, 32, 64), _F32), (nb, 32, 32, 64),
            [pltpu.VMEM((nb, 34, 32, 192), _F32), pltpu.VMEM((nb, 34, 32, 192), _F32)])

        nb = 8
        h2 = _run_stage(
            functools.partial(_down_body, nb=nb, h=32, cin=64, cout=128), nloc // nb,
            [h1, w2d, b2d, w21a, b21a, w21b, b21b, w22a, b22a, w22b, b22b],
            [(nb, 32, 32, 64)] + [None] * 10,
            jax.ShapeDtypeStruct((nloc, 16, 16, 128), _F32), (nb, 16, 16, 128),
            [pltpu.VMEM((2, nb, 17, 16, 192), _F32), pltpu.VMEM((nb, 18, 16, 384), _F32)])

        nb = 16
        h3 = _run_stage(
            functools.partial(_down_body, nb=nb, h=16, cin=128, cout=256), nloc // nb,
            [h2, w3d, b3d, w31a, b31a, w31b, b31b, w32a, b32a, w32b, b32b],
            [(nb, 16, 16, 128)] + [None] * 10,
            jax.ShapeDtypeStruct((nloc, 8, 8, 256), _F32), (nb, 8, 8, 256),
            [pltpu.VMEM((2, nb, 9, 8, 384), _F32), pltpu.VMEM((nb, 10, 8, 768), _F32)])

        nb = 16
        h4 = _run_stage(
            functools.partial(_l4b0_body, nb=nb), nloc // nb,
            [h3, w4d, b4d, w41a, b41a, w41b, b41b],
            [(nb, 8, 8, 256)] + [None] * 6,
            jax.ShapeDtypeStruct((nloc, 4, 4, 512), _F32), (nb, 4, 4, 512),
            [pltpu.VMEM((2, nb, 5, 4, 768), _F32), pltpu.VMEM((nb, 6, 4, 1536), _F32)])

        nb = 16
        out = _run_stage(
            functools.partial(_l4b1_head_body, nb=nb), nloc // nb,
            [h4, w42a, b42a, w42b, b42b, wfc, bfc],
            [(nb, 4, 4, 512)] + [None] * 6,
            jax.ShapeDtypeStruct((nloc, ncls_p), _F32), (nb, ncls_p),
            [pltpu.VMEM((nb, 6, 4, 1536), _F32)])
        return out

    ws = (w0, b0, w11a, b11a, w11b, b11b, w12a, b12a, w12b, b12b,
          w2d, b2d, w21a, b21a, w21b, b21b, w22a, b22a, w22b, b22b,
          w3d, b3d, w31a, b31a, w31b, b31b, w32a, b32a, w32b, b32b,
          w4d, b4d, w41a, b41a, w41b, b41b,
          w42a, b42a, w42b, b42b, wfc, bfc)
    devs = jax.devices()
    ndev = 2 if len(devs) >= 2 and n % 32 == 0 else 1
    if ndev > 1:
        mesh = Mesh(np.array(devs[:ndev]), ("b",))
        sharded = shard_map(_forward, mesh=mesh,
                            in_specs=(P("b"),) + (P(),) * len(ws),
                            out_specs=P("b"), check_rep=False)
        out = sharded(x27, *ws)
        out = jax.lax.with_sharding_constraint(out, NamedSharding(mesh, P()))
    else:
        out = _forward(x27, *ws)
    return out[:, :ncls]
